# baseline (speedup 1.0000x reference)
import jax
import jax.numpy as jnp
from jax.experimental import pallas as pl

SC = 2
CB = 16


def _upsample_block(x_ref, o_ref):
    xb = x_ref[...]
    cb, h, w = xb.shape
    oh, ow = SC * h, SC * w
    li = jax.lax.broadcasted_iota(jnp.int32, (cb, h, ow), 2)
    src = jnp.maximum(li - 1, 0) // 2
    cols = jnp.where(li % 2 == 1, jnp.take_along_axis(xb, src, axis=2), 0.0)
    zrows = jnp.zeros_like(cols)
    o_ref[...] = jnp.stack([zrows, cols], axis=2).reshape(cb, oh, ow)


def kernel(x):
    B, I, C, H, W = x.shape
    n = B * I * C
    xf = x.reshape(n, H, W)
    out = pl.pallas_call(
        _upsample_block,
        grid=(n // CB,),
        in_specs=[pl.BlockSpec((CB, H, W), lambda i: (i, 0, 0))],
        out_specs=pl.BlockSpec((CB, SC * H, SC * W), lambda i: (i, 0, 0)),
        out_shape=jax.ShapeDtypeStruct((n, SC * H, SC * W), x.dtype),
    )(xf)
    return out.reshape(B, I, C, SC * H, SC * W)
